# baseline (speedup 1.0000x reference)
import functools

import jax
import jax.numpy as jnp
from jax import lax
from jax.experimental import pallas as pl
from jax.experimental.pallas import tpu as pltpu
from jax.experimental.pallas import tpu_sc as plsc

_NT = 25
_WIND_W = 4
_WIND_IDX = 10
_WSPM_MEAN = 2.5
_WSPM_SCALE = 1.8
_MAX_DELAY = 24.0
_N = 128
_F = 32
_CI = 64


def _sc_body(xs_hbm, xw_hbm, adj_hbm, dist_hbm, out_hbm,
             xs_v, wind_v, invs_v, adj_v, dist_v, trow_v, w0row_v, w1row_v,
             out_v, xs_sem):
    nc = 2
    b = lax.axis_index("s") * nc + lax.axis_index("c")

    xs_cp = pltpu.make_async_copy(xs_hbm.at[b], xs_v, xs_sem)
    xs_cp.start()

    pltpu.sync_copy(xw_hbm.at[b], wind_v)
    for k in range(_N // 16):
        sl = pl.ds(16 * k, 16)
        acc = jnp.zeros((16,), jnp.float32)
        for t in range(_WIND_W):
            acc = acc + wind_v[t, sl]
        wspm = jnp.maximum(acc * (1.0 / _WIND_W) * _WSPM_SCALE + _WSPM_MEAN, 0.0)
        invs_v[sl] = 1.0 / (wspm * 3.6 + 0.001)

    iota = lax.iota(jnp.int32, 16)
    first = True
    for c in range(_N // _CI):
        pltpu.sync_copy(adj_hbm.at[b, pl.ds(c * _CI, _CI)], adj_v)
        pltpu.sync_copy(dist_hbm.at[pl.ds(c * _CI, _CI)], dist_v)
        if first:
            xs_cp.wait()
            first = False

        def row_body(ii, _):
            for k in range(_N // 16):
                sl = pl.ds(16 * k, 16)
                tau = jnp.minimum(dist_v[ii, sl] * invs_v[sl], _MAX_DELAY)
                tq = (_NT - 1.0) - tau
                t0i = jnp.minimum(tq.astype(jnp.int32), _NT - 2)
                w1 = tq - t0i.astype(jnp.float32)
                a = adj_v[ii, sl]
                w1a = a * w1
                trow_v[sl] = t0i * (_N * _F)
                w0row_v[sl] = a - w1a
                w1row_v[sl] = w1a

            def blk_body(k, carry):
                acc0, acc1 = carry
                sl = pl.ds(16 * k, 16)
                t0blk = trow_v[sl]
                w0blk = w0row_v[sl]
                w1blk = w1row_v[sl]
                kbase = (16 * k) * _F
                for jj in range(16):
                    lane = jnp.full((16,), jj, jnp.int32)
                    t0b = jnp.take(t0blk, lane)
                    w0b = jnp.take(w0blk, lane)
                    w1b = jnp.take(w1blk, lane)
                    i00 = t0b + (kbase + jj * _F) + iota
                    i01 = i00 + 16
                    i10 = i00 + (_N * _F)
                    i11 = i01 + (_N * _F)
                    x00 = plsc.load_gather(xs_v, [i00])
                    x01 = plsc.load_gather(xs_v, [i01])
                    x10 = plsc.load_gather(xs_v, [i10])
                    x11 = plsc.load_gather(xs_v, [i11])
                    acc0 = acc0 + w0b * x00 + w1b * x10
                    acc1 = acc1 + w0b * x01 + w1b * x11
                return acc0, acc1

            z = jnp.zeros((16,), jnp.float32)
            acc0, acc1 = lax.fori_loop(0, _N // 16, blk_body, (z, z))

            ro = (c * _CI + ii) * _F
            out_v[pl.ds(ro, 16)] = acc0
            out_v[pl.ds(ro + 16, 16)] = acc1
            return 0

        lax.fori_loop(0, _CI, row_body, 0)

    pltpu.sync_copy(out_v, out_hbm.at[b])


def kernel(x_raw, adj, dist_km):
    B, T, N, F = x_raw.shape
    assert (B, N, F) == (32, _N, _F)
    xs = lax.slice_in_dim(x_raw, T - _NT, T, axis=1)
    xs = xs.reshape(B, _NT * N * F)
    xw = x_raw[:, T - _WIND_W:, :, _WIND_IDX]
    mesh = plsc.VectorSubcoreMesh(core_axis_name="c", subcore_axis_name="s")
    run = functools.partial(
        pl.kernel,
        out_type=jax.ShapeDtypeStruct((B, N * F), jnp.float32),
        mesh=mesh,
        compiler_params=pltpu.CompilerParams(use_tc_tiling_on_sc=False, needs_layout_passes=False),
        scratch_types=[
            pltpu.VMEM((_NT * _N * _F,), jnp.float32),
            pltpu.VMEM((_WIND_W, _N), jnp.float32),
            pltpu.VMEM((_N,), jnp.float32),
            pltpu.VMEM((_CI, _N), jnp.float32),
            pltpu.VMEM((_CI, _N), jnp.float32),
            pltpu.VMEM((_N,), jnp.int32),
            pltpu.VMEM((_N,), jnp.float32),
            pltpu.VMEM((_N,), jnp.float32),
            pltpu.VMEM((_N * _F,), jnp.float32),
            pltpu.SemaphoreType.DMA,
        ],
    )(_sc_body)
    out = run(xs, xw, adj, dist_km)
    return out.reshape(B, N, F)
